# Initial kernel scaffold; baseline (speedup 1.0000x reference)
import jax
import jax.numpy as jnp
from jax.experimental import pallas as pl


def kernel(x, edge_index, W1_l, W1_r, b1, W2_l, W2_r, b2):
    raise NotImplementedError("write your pallas kernel here")


# speedup vs baseline: 7.3214x; 7.3214x over previous
import functools

import jax
import jax.numpy as jnp
from jax import lax
from jax.experimental import pallas as pl
from jax.experimental.pallas import tpu as pltpu
from jax.experimental.pallas import tpu_sc as plsc

N = 10000
E = 320000
D = 128
DP = 144
NC = 2
NS = 16
NW = NC * NS
EPW = E // NW
C = 80
STEPS = EPW // C
NPAD = 10112
RPT = NPAD // NS


def _make_segsum(dp):
  mesh = plsc.VectorSubcoreMesh(
      core_axis_name="c", subcore_axis_name="s", num_cores=NC, num_subcores=NS)

  @functools.partial(
      pl.kernel,
      mesh=mesh,
      out_type=jax.ShapeDtypeStruct((NC, NPAD, dp), jnp.float32),
      scratch_types=[
          pltpu.VMEM((STEPS, C), jnp.int32),
          pltpu.VMEM((STEPS, C), jnp.int32),
          pltpu.VMEM((C, dp), jnp.float32),
          pltpu.VMEM_SHARED((NPAD, dp), jnp.float32),
          pltpu.SemaphoreType.DMA,
      ],
      compiler_params=pltpu.CompilerParams(use_tc_tiling_on_sc=False),
  )
  def seg(table, srcw, dstw, zeros, out, src_v, dst_v, rows_v, acc_sh, gsem):
    cid = lax.axis_index("c")
    sid = lax.axis_index("s")
    wid = cid * NS + sid
    pltpu.sync_copy(zeros, acc_sh.at[pl.ds(sid * RPT, RPT)])
    pltpu.sync_copy(srcw.at[wid], src_v)
    pltpu.sync_copy(dstw.at[wid], dst_v)
    plsc.subcore_barrier()

    def body(s, carry):
      pltpu.async_copy(table.at[src_v.at[s]], rows_v, gsem).wait()
      pltpu.sync_copy(rows_v, acc_sh.at[dst_v.at[s]], add=True)
      return carry

    lax.fori_loop(0, STEPS, body, 0)
    plsc.subcore_barrier()
    pltpu.sync_copy(acc_sh.at[pl.ds(sid * RPT, RPT)],
                    out.at[cid, pl.ds(sid * RPT, RPT)])

  return seg


_segsum_l1 = _make_segsum(DP)
_segsum_l2 = _make_segsum(D)

_BN = 1000


def _dense1_body(acc_ref, x_ref, wl_ref, wr_ref, b_ref, h_ref, dv_ref):
  s = acc_ref[0] + acc_ref[1]
  deg = s[:, D:D + 1]
  dinv = 1.0 / jnp.maximum(deg, 1.0)
  agg = s[:, :D] * dinv
  h = lax.dot_general(agg, wl_ref[...], (((1,), (1,)), ((), ())),
                      preferred_element_type=jnp.float32)
  h = h + lax.dot_general(x_ref[...], wr_ref[...], (((1,), (1,)), ((), ())),
                          preferred_element_type=jnp.float32)
  h = h + b_ref[...]
  h_ref[...] = jnp.maximum(h, 0.0)
  dv_ref[...] = jnp.broadcast_to(dinv, (_BN, D))


def _dense2_body(acc_ref, dv_ref, h_ref, wl_ref, wr_ref, b_ref, o_ref):
  agg = (acc_ref[0] + acc_ref[1]) * dv_ref[...]
  o = lax.dot_general(agg, wl_ref[...], (((1,), (1,)), ((), ())),
                      preferred_element_type=jnp.float32)
  o = o + lax.dot_general(h_ref[...], wr_ref[...], (((1,), (1,)), ((), ())),
                          preferred_element_type=jnp.float32)
  o_ref[...] = o + b_ref[...]


def _dense1(acc, x, wl, wr, b):
  grid = (N // _BN,)
  return pl.pallas_call(
      _dense1_body,
      grid=grid,
      in_specs=[
          pl.BlockSpec((NC, _BN, DP), lambda i: (0, i, 0)),
          pl.BlockSpec((_BN, D), lambda i: (i, 0)),
          pl.BlockSpec((D, D), lambda i: (0, 0)),
          pl.BlockSpec((D, D), lambda i: (0, 0)),
          pl.BlockSpec((1, D), lambda i: (0, 0)),
      ],
      out_specs=[
          pl.BlockSpec((_BN, D), lambda i: (i, 0)),
          pl.BlockSpec((_BN, D), lambda i: (i, 0)),
      ],
      out_shape=[
          jax.ShapeDtypeStruct((N, D), jnp.float32),
          jax.ShapeDtypeStruct((N, D), jnp.float32),
      ],
  )(acc, x, wl, wr, b)


def _dense2(acc, dv, h, wl, wr, b):
  grid = (N // _BN,)
  return pl.pallas_call(
      _dense2_body,
      grid=grid,
      in_specs=[
          pl.BlockSpec((NC, _BN, D), lambda i: (0, i, 0)),
          pl.BlockSpec((_BN, D), lambda i: (i, 0)),
          pl.BlockSpec((_BN, D), lambda i: (i, 0)),
          pl.BlockSpec((D, D), lambda i: (0, 0)),
          pl.BlockSpec((D, D), lambda i: (0, 0)),
          pl.BlockSpec((1, D), lambda i: (0, 0)),
      ],
      out_specs=pl.BlockSpec((_BN, D), lambda i: (i, 0)),
      out_shape=jax.ShapeDtypeStruct((N, D), jnp.float32),
  )(acc, dv, h, wl, wr, b)


def kernel(x, edge_index, W1_l, W1_r, b1, W2_l, W2_r, b2):
  src = edge_index[0].reshape(NW, STEPS, C)
  dst = edge_index[1].reshape(NW, STEPS, C)
  xpad = jnp.concatenate(
      [x, jnp.ones((N, 1), jnp.float32), jnp.zeros((N, DP - D - 1), jnp.float32)],
      axis=1)
  zeros_dp = jnp.zeros((RPT, DP), jnp.float32)
  zeros_d = jnp.zeros((RPT, D), jnp.float32)

  acc1 = _segsum_l1(xpad, src, dst, zeros_dp)
  h, dv = _dense1(acc1, x, W1_l, W1_r, b1.reshape(1, D))
  acc2 = _segsum_l2(h, src, dst, zeros_d)
  out = _dense2(acc2, dv, h, W2_l, W2_r, b2.reshape(1, D))
  return out


# speedup vs baseline: 10.6633x; 1.4565x over previous
import functools

import jax
import jax.numpy as jnp
from jax import lax
from jax.experimental import pallas as pl
from jax.experimental.pallas import tpu as pltpu
from jax.experimental.pallas import tpu_sc as plsc

N = 10000
E = 320000
D = 128
DP = 144
NC = 2
NS = 16
NW = NC * NS
EPW = E // NW
C = 100
STEPS = EPW // C
SB = 10
NB = STEPS // SB
NPAD = 10112
RPT = NPAD // NS


def _make_segsum(dp):
  mesh = plsc.VectorSubcoreMesh(
      core_axis_name="c", subcore_axis_name="s", num_cores=NC, num_subcores=NS)

  @functools.partial(
      pl.kernel,
      mesh=mesh,
      out_type=jax.ShapeDtypeStruct((NC, NPAD, dp), jnp.float32),
      scratch_types=[
          pltpu.VMEM((2, SB, 2, C), jnp.int32),
          pltpu.VMEM((2, C, dp), jnp.float32),
          pltpu.VMEM_SHARED((NPAD, dp), jnp.float32),
          pltpu.SemaphoreType.DMA,
          pltpu.SemaphoreType.DMA,
          pltpu.SemaphoreType.DMA,
      ],
      compiler_params=pltpu.CompilerParams(use_tc_tiling_on_sc=False),
  )
  def seg(table, idxw, zeros, out, idx_v, rows_v, acc_sh, gsem0, gsem1, isem):
    cid = lax.axis_index("c")
    sid = lax.axis_index("s")
    wid = cid * NS + sid
    pltpu.sync_copy(zeros, acc_sh.at[pl.ds(sid * RPT, RPT)])
    pltpu.sync_copy(idxw.at[wid, 0], idx_v.at[0])
    plsc.subcore_barrier()

    pltpu.async_copy(table.at[idx_v.at[0, 0, 0]], rows_v.at[0], gsem0)

    def block(b, carry):
      bp = b % 2

      @pl.when(b + 1 < NB)
      def _():
        pltpu.async_copy(idxw.at[wid, b + 1], idx_v.at[1 - bp], isem)

      def pair(j, carry2):
        j0 = 2 * j
        j1 = j0 + 1
        pltpu.async_copy(table.at[idx_v.at[bp, j1, 0]], rows_v.at[1], gsem1)
        pltpu.make_async_copy(table.at[idx_v.at[bp, j0, 0]], rows_v.at[0],
                              gsem0).wait()
        pltpu.sync_copy(rows_v.at[0], acc_sh.at[idx_v.at[bp, j0, 1]], add=True)

        @pl.when(j1 + 1 < SB)
        def _():
          pltpu.async_copy(table.at[idx_v.at[bp, j0 + 2, 0]], rows_v.at[0],
                           gsem0)

        @pl.when((j1 + 1 >= SB) & (b + 1 < NB))
        def _():
          pltpu.make_async_copy(idxw.at[wid, b + 1], idx_v.at[1 - bp],
                                isem).wait()
          pltpu.async_copy(table.at[idx_v.at[1 - bp, 0, 0]], rows_v.at[0],
                           gsem0)

        pltpu.make_async_copy(table.at[idx_v.at[bp, j1, 0]], rows_v.at[1],
                              gsem1).wait()
        pltpu.sync_copy(rows_v.at[1], acc_sh.at[idx_v.at[bp, j1, 1]], add=True)
        return carry2

      lax.fori_loop(0, SB // 2, pair, 0)
      return carry

    lax.fori_loop(0, NB, block, 0)
    plsc.subcore_barrier()
    pltpu.sync_copy(acc_sh.at[pl.ds(sid * RPT, RPT)],
                    out.at[cid, pl.ds(sid * RPT, RPT)])

  return seg


_segsum_l1 = _make_segsum(DP)
_segsum_l2 = _make_segsum(D)

_BN = 1000


def _dense1_body(acc_ref, x_ref, wl_ref, wr_ref, b_ref, h_ref, dv_ref):
  s = acc_ref[0] + acc_ref[1]
  deg = s[:, D:D + 1]
  dinv = 1.0 / jnp.maximum(deg, 1.0)
  agg = s[:, :D] * dinv
  h = lax.dot_general(agg, wl_ref[...], (((1,), (1,)), ((), ())),
                      preferred_element_type=jnp.float32)
  h = h + lax.dot_general(x_ref[...], wr_ref[...], (((1,), (1,)), ((), ())),
                          preferred_element_type=jnp.float32)
  h = h + b_ref[...]
  h_ref[...] = jnp.maximum(h, 0.0)
  dv_ref[...] = jnp.broadcast_to(dinv, (_BN, D))


def _dense2_body(acc_ref, dv_ref, h_ref, wl_ref, wr_ref, b_ref, o_ref):
  agg = (acc_ref[0] + acc_ref[1]) * dv_ref[...]
  o = lax.dot_general(agg, wl_ref[...], (((1,), (1,)), ((), ())),
                      preferred_element_type=jnp.float32)
  o = o + lax.dot_general(h_ref[...], wr_ref[...], (((1,), (1,)), ((), ())),
                          preferred_element_type=jnp.float32)
  o_ref[...] = o + b_ref[...]


def _dense1(acc, x, wl, wr, b):
  grid = (N // _BN,)
  return pl.pallas_call(
      _dense1_body,
      grid=grid,
      in_specs=[
          pl.BlockSpec((NC, _BN, DP), lambda i: (0, i, 0)),
          pl.BlockSpec((_BN, D), lambda i: (i, 0)),
          pl.BlockSpec((D, D), lambda i: (0, 0)),
          pl.BlockSpec((D, D), lambda i: (0, 0)),
          pl.BlockSpec((1, D), lambda i: (0, 0)),
      ],
      out_specs=[
          pl.BlockSpec((_BN, D), lambda i: (i, 0)),
          pl.BlockSpec((_BN, D), lambda i: (i, 0)),
      ],
      out_shape=[
          jax.ShapeDtypeStruct((N, D), jnp.float32),
          jax.ShapeDtypeStruct((N, D), jnp.float32),
      ],
  )(acc, x, wl, wr, b)


def _dense2(acc, dv, h, wl, wr, b):
  grid = (N // _BN,)
  return pl.pallas_call(
      _dense2_body,
      grid=grid,
      in_specs=[
          pl.BlockSpec((NC, _BN, D), lambda i: (0, i, 0)),
          pl.BlockSpec((_BN, D), lambda i: (i, 0)),
          pl.BlockSpec((_BN, D), lambda i: (i, 0)),
          pl.BlockSpec((D, D), lambda i: (0, 0)),
          pl.BlockSpec((D, D), lambda i: (0, 0)),
          pl.BlockSpec((1, D), lambda i: (0, 0)),
      ],
      out_specs=pl.BlockSpec((_BN, D), lambda i: (i, 0)),
      out_shape=jax.ShapeDtypeStruct((N, D), jnp.float32),
  )(acc, dv, h, wl, wr, b)


def kernel(x, edge_index, W1_l, W1_r, b1, W2_l, W2_r, b2):
  src = edge_index[0].reshape(NW, NB, SB, 1, C)
  dst = edge_index[1].reshape(NW, NB, SB, 1, C)
  idxw = jnp.concatenate([src, dst], axis=3)
  xpad = jnp.concatenate(
      [x, jnp.ones((N, 1), jnp.float32), jnp.zeros((N, DP - D - 1), jnp.float32)],
      axis=1)
  zeros_dp = jnp.zeros((RPT, DP), jnp.float32)
  zeros_d = jnp.zeros((RPT, D), jnp.float32)

  acc1 = _segsum_l1(xpad, idxw, zeros_dp)
  h, dv = _dense1(acc1, x, W1_l, W1_r, b1.reshape(1, D))
  acc2 = _segsum_l2(h, idxw, zeros_d)
  out = _dense2(acc2, dv, h, W2_l, W2_r, b2.reshape(1, D))
  return out


# speedup vs baseline: 12.4786x; 1.1702x over previous
import functools

import jax
import jax.numpy as jnp
from jax import lax
from jax.experimental import pallas as pl
from jax.experimental.pallas import tpu as pltpu
from jax.experimental.pallas import tpu_sc as plsc

N = 10000
E = 320000
D = 128
DG = 16
NC = 2
NS = 16
NW = NC * NS
EPW = E // NW
C = 100
STEPS = EPW // C
SB = 10
NB = STEPS // SB
NPAD = 10112
RPT = NPAD // NS


def _make_segsum(with_deg):
  mesh = plsc.VectorSubcoreMesh(
      core_axis_name="c", subcore_axis_name="s", num_cores=NC, num_subcores=NS)

  acc_t = jax.ShapeDtypeStruct((NC, NPAD, D), jnp.float32)
  out_type = [acc_t, jax.ShapeDtypeStruct((NC, NPAD, DG), jnp.float32)
              ] if with_deg else acc_t
  scratch = [
      pltpu.VMEM((2, SB, C), jnp.int32),
      pltpu.VMEM((2, SB, C), jnp.int32),
      pltpu.VMEM((2, C, D), jnp.float32),
      pltpu.VMEM_SHARED((NPAD, D), jnp.float32),
      pltpu.SemaphoreType.DMA,
      pltpu.SemaphoreType.DMA,
      pltpu.SemaphoreType.DMA,
  ]
  if with_deg:
    scratch += [
        pltpu.VMEM((C, DG), jnp.float32),
        pltpu.VMEM_SHARED((NPAD, DG), jnp.float32),
        pltpu.SemaphoreType.DMA,
    ]

  @functools.partial(
      pl.kernel,
      mesh=mesh,
      out_type=out_type,
      scratch_types=scratch,
      compiler_params=pltpu.CompilerParams(use_tc_tiling_on_sc=False),
  )
  def seg(table, srcw, dstw, zeros, *rest):
    if with_deg:
      (ones, zeros_dg, out, out_dg, src_v, dst_v, rows_v, acc_sh,
       gsem0, gsem1, isem, ones_v, deg_sh, dsem) = rest
    else:
      out, src_v, dst_v, rows_v, acc_sh, gsem0, gsem1, isem = rest
    cid = lax.axis_index("c")
    sid = lax.axis_index("s")
    wid = cid * NS + sid
    pltpu.sync_copy(zeros, acc_sh.at[pl.ds(sid * RPT, RPT)])
    if with_deg:
      pltpu.sync_copy(zeros_dg, deg_sh.at[pl.ds(sid * RPT, RPT)])
      pltpu.sync_copy(ones, ones_v)
    pltpu.sync_copy(srcw.at[wid, 0], src_v.at[0])
    pltpu.sync_copy(dstw.at[wid, 0], dst_v.at[0])
    plsc.subcore_barrier()

    pltpu.async_copy(table.at[src_v.at[0, 0]], rows_v.at[0], gsem0)

    def scat(bp, j, buf, sem):
      pltpu.make_async_copy(table.at[src_v.at[bp, j]], rows_v.at[buf],
                            sem).wait()
      pltpu.sync_copy(rows_v.at[buf], acc_sh.at[dst_v.at[bp, j]], add=True)
      if with_deg:
        pltpu.async_copy(ones_v, deg_sh.at[dst_v.at[bp, j]], dsem, add=True)

    def block(b, carry):
      bp = b % 2

      @pl.when(b + 1 < NB)
      def _():
        pltpu.async_copy(srcw.at[wid, b + 1], src_v.at[1 - bp], isem)
        pltpu.async_copy(dstw.at[wid, b + 1], dst_v.at[1 - bp], isem)

      def pair(j, carry2):
        j0 = 2 * j
        j1 = j0 + 1
        pltpu.async_copy(table.at[src_v.at[bp, j1]], rows_v.at[1], gsem1)
        scat(bp, j0, 0, gsem0)

        @pl.when(j1 + 1 < SB)
        def _():
          pltpu.async_copy(table.at[src_v.at[bp, j0 + 2]], rows_v.at[0], gsem0)

        @pl.when((j1 + 1 >= SB) & (b + 1 < NB))
        def _():
          pltpu.make_async_copy(srcw.at[wid, b + 1], src_v.at[1 - bp],
                                isem).wait()
          pltpu.make_async_copy(dstw.at[wid, b + 1], dst_v.at[1 - bp],
                                isem).wait()
          pltpu.async_copy(table.at[src_v.at[1 - bp, 0]], rows_v.at[0], gsem0)

        scat(bp, j1, 1, gsem1)
        return carry2

      lax.fori_loop(0, SB // 2, pair, 0)
      return carry

    lax.fori_loop(0, NB, block, 0)
    if with_deg:
      def drain(s, carry):
        pltpu.make_async_copy(ones_v, deg_sh.at[dst_v.at[0, 0]], dsem).wait()
        return carry
      lax.fori_loop(0, STEPS, drain, 0)
    plsc.subcore_barrier()
    pltpu.sync_copy(acc_sh.at[pl.ds(sid * RPT, RPT)],
                    out.at[cid, pl.ds(sid * RPT, RPT)])
    if with_deg:
      pltpu.sync_copy(deg_sh.at[pl.ds(sid * RPT, RPT)],
                      out_dg.at[cid, pl.ds(sid * RPT, RPT)])

  return seg


_segsum_l1 = _make_segsum(True)
_segsum_l2 = _make_segsum(False)

_BN = 1000


def _dense1_body(acc_ref, dg_ref, x_ref, wl_ref, wr_ref, b_ref, h_ref, dv_ref):
  s = acc_ref[0] + acc_ref[1]
  deg = dg_ref[0, :, 0:1] + dg_ref[1, :, 0:1]
  dinv = 1.0 / jnp.maximum(deg, 1.0)
  agg = s * dinv
  h = lax.dot_general(agg, wl_ref[...], (((1,), (1,)), ((), ())),
                      preferred_element_type=jnp.float32)
  h = h + lax.dot_general(x_ref[...], wr_ref[...], (((1,), (1,)), ((), ())),
                          preferred_element_type=jnp.float32)
  h = h + b_ref[...]
  h_ref[...] = jnp.maximum(h, 0.0)
  dv_ref[...] = jnp.broadcast_to(dinv, (_BN, D))


def _dense2_body(acc_ref, dv_ref, h_ref, wl_ref, wr_ref, b_ref, o_ref):
  agg = (acc_ref[0] + acc_ref[1]) * dv_ref[...]
  o = lax.dot_general(agg, wl_ref[...], (((1,), (1,)), ((), ())),
                      preferred_element_type=jnp.float32)
  o = o + lax.dot_general(h_ref[...], wr_ref[...], (((1,), (1,)), ((), ())),
                          preferred_element_type=jnp.float32)
  o_ref[...] = o + b_ref[...]


def _dense1(acc, dg, x, wl, wr, b):
  grid = (N // _BN,)
  return pl.pallas_call(
      _dense1_body,
      grid=grid,
      in_specs=[
          pl.BlockSpec((NC, _BN, D), lambda i: (0, i, 0)),
          pl.BlockSpec((NC, _BN, DG), lambda i: (0, i, 0)),
          pl.BlockSpec((_BN, D), lambda i: (i, 0)),
          pl.BlockSpec((D, D), lambda i: (0, 0)),
          pl.BlockSpec((D, D), lambda i: (0, 0)),
          pl.BlockSpec((1, D), lambda i: (0, 0)),
      ],
      out_specs=[
          pl.BlockSpec((_BN, D), lambda i: (i, 0)),
          pl.BlockSpec((_BN, D), lambda i: (i, 0)),
      ],
      out_shape=[
          jax.ShapeDtypeStruct((N, D), jnp.float32),
          jax.ShapeDtypeStruct((N, D), jnp.float32),
      ],
  )(acc, dg, x, wl, wr, b)


def _dense2(acc, dv, h, wl, wr, b):
  grid = (N // _BN,)
  return pl.pallas_call(
      _dense2_body,
      grid=grid,
      in_specs=[
          pl.BlockSpec((NC, _BN, D), lambda i: (0, i, 0)),
          pl.BlockSpec((_BN, D), lambda i: (i, 0)),
          pl.BlockSpec((_BN, D), lambda i: (i, 0)),
          pl.BlockSpec((D, D), lambda i: (0, 0)),
          pl.BlockSpec((D, D), lambda i: (0, 0)),
          pl.BlockSpec((1, D), lambda i: (0, 0)),
      ],
      out_specs=pl.BlockSpec((_BN, D), lambda i: (i, 0)),
      out_shape=jax.ShapeDtypeStruct((N, D), jnp.float32),
  )(acc, dv, h, wl, wr, b)


def kernel(x, edge_index, W1_l, W1_r, b1, W2_l, W2_r, b2):
  src = edge_index[0].reshape(NW, NB, SB, C)
  dst = edge_index[1].reshape(NW, NB, SB, C)
  zeros_d = jnp.zeros((RPT, D), jnp.float32)
  zeros_dg = jnp.zeros((RPT, DG), jnp.float32)
  ones_c = jnp.ones((C, DG), jnp.float32)

  acc1, dg = _segsum_l1(x, src, dst, zeros_d, ones_c, zeros_dg)
  h, dv = _dense1(acc1, dg, x, W1_l, W1_r, b1.reshape(1, D))
  acc2 = _segsum_l2(h, src, dst, zeros_d)
  out = _dense2(acc2, dv, h, W2_l, W2_r, b2.reshape(1, D))
  return out


# speedup vs baseline: 13.1488x; 1.0537x over previous
import functools

import jax
import jax.numpy as jnp
from jax import lax
from jax.experimental import pallas as pl
from jax.experimental.pallas import tpu as pltpu
from jax.experimental.pallas import tpu_sc as plsc

N = 10000
E = 320000
D = 128
DG = 16
NC = 2
NS = 16
NW = NC * NS
EPW = E // NW
C = 125
STEPS = EPW // C
SB = 10
NB = STEPS // SB
NPAD = 10112
RPT = NPAD // NS


def _make_segsum(with_deg):
  mesh = plsc.VectorSubcoreMesh(
      core_axis_name="c", subcore_axis_name="s", num_cores=NC, num_subcores=NS)

  acc_t = jax.ShapeDtypeStruct((NC, NPAD, D), jnp.float32)
  out_type = [acc_t, jax.ShapeDtypeStruct((NC, NPAD, DG), jnp.float32)
              ] if with_deg else acc_t
  scratch = [
      pltpu.VMEM((2, SB, C), jnp.int32),
      pltpu.VMEM((2, SB, C), jnp.int32),
      pltpu.VMEM((2, C, D), jnp.float32),
      pltpu.VMEM_SHARED((NPAD, D), jnp.float32),
      pltpu.SemaphoreType.DMA,
      pltpu.SemaphoreType.DMA,
      pltpu.SemaphoreType.DMA,
  ]
  if with_deg:
    scratch += [
        pltpu.VMEM((C, DG), jnp.float32),
        pltpu.VMEM_SHARED((NPAD, DG), jnp.float32),
        pltpu.SemaphoreType.DMA,
    ]

  @functools.partial(
      pl.kernel,
      mesh=mesh,
      out_type=out_type,
      scratch_types=scratch,
      compiler_params=pltpu.CompilerParams(use_tc_tiling_on_sc=False),
  )
  def seg(table, srcw, dstw, zeros, *rest):
    if with_deg:
      (ones, zeros_dg, out, out_dg, src_v, dst_v, rows_v, acc_sh,
       gsem0, gsem1, isem, ones_v, deg_sh, dsem) = rest
    else:
      out, src_v, dst_v, rows_v, acc_sh, gsem0, gsem1, isem = rest
    cid = lax.axis_index("c")
    sid = lax.axis_index("s")
    wid = cid * NS + sid
    pltpu.sync_copy(zeros, acc_sh.at[pl.ds(sid * RPT, RPT)])
    if with_deg:
      pltpu.sync_copy(zeros_dg, deg_sh.at[pl.ds(sid * RPT, RPT)])
      pltpu.sync_copy(ones, ones_v)
    pltpu.sync_copy(srcw.at[wid, 0], src_v.at[0])
    pltpu.sync_copy(dstw.at[wid, 0], dst_v.at[0])
    plsc.subcore_barrier()

    pltpu.async_copy(table.at[src_v.at[0, 0]], rows_v.at[0], gsem0)

    def scat(bp, j, buf, sem):
      pltpu.make_async_copy(table.at[src_v.at[bp, j]], rows_v.at[buf],
                            sem).wait()
      pltpu.sync_copy(rows_v.at[buf], acc_sh.at[dst_v.at[bp, j]], add=True)
      if with_deg:
        pltpu.async_copy(ones_v, deg_sh.at[dst_v.at[bp, j]], dsem, add=True)

    def block(b, carry):
      bp = b % 2

      @pl.when(b + 1 < NB)
      def _():
        pltpu.async_copy(srcw.at[wid, b + 1], src_v.at[1 - bp], isem)
        pltpu.async_copy(dstw.at[wid, b + 1], dst_v.at[1 - bp], isem)

      def pair(j, carry2):
        j0 = 2 * j
        j1 = j0 + 1
        pltpu.async_copy(table.at[src_v.at[bp, j1]], rows_v.at[1], gsem1)
        scat(bp, j0, 0, gsem0)

        @pl.when(j1 + 1 < SB)
        def _():
          pltpu.async_copy(table.at[src_v.at[bp, j0 + 2]], rows_v.at[0], gsem0)

        @pl.when((j1 + 1 >= SB) & (b + 1 < NB))
        def _():
          pltpu.make_async_copy(srcw.at[wid, b + 1], src_v.at[1 - bp],
                                isem).wait()
          pltpu.make_async_copy(dstw.at[wid, b + 1], dst_v.at[1 - bp],
                                isem).wait()
          pltpu.async_copy(table.at[src_v.at[1 - bp, 0]], rows_v.at[0], gsem0)

        scat(bp, j1, 1, gsem1)
        return carry2

      lax.fori_loop(0, SB // 2, pair, 0)
      return carry

    lax.fori_loop(0, NB, block, 0)
    if with_deg:
      def drain(s, carry):
        pltpu.make_async_copy(ones_v, deg_sh.at[dst_v.at[0, 0]], dsem).wait()
        return carry
      lax.fori_loop(0, STEPS, drain, 0)
    plsc.subcore_barrier()
    pltpu.sync_copy(acc_sh.at[pl.ds(sid * RPT, RPT)],
                    out.at[cid, pl.ds(sid * RPT, RPT)])
    if with_deg:
      pltpu.sync_copy(deg_sh.at[pl.ds(sid * RPT, RPT)],
                      out_dg.at[cid, pl.ds(sid * RPT, RPT)])

  return seg


_segsum_l1 = _make_segsum(True)
_segsum_l2 = _make_segsum(False)

_BN = 1000


def _dense1_body(acc_ref, dg_ref, x_ref, wl_ref, wr_ref, b_ref, h_ref, dv_ref):
  s = acc_ref[0] + acc_ref[1]
  deg = dg_ref[0, :, 0:1] + dg_ref[1, :, 0:1]
  dinv = 1.0 / jnp.maximum(deg, 1.0)
  agg = s * dinv
  h = lax.dot_general(agg, wl_ref[...], (((1,), (1,)), ((), ())),
                      preferred_element_type=jnp.float32)
  h = h + lax.dot_general(x_ref[...], wr_ref[...], (((1,), (1,)), ((), ())),
                          preferred_element_type=jnp.float32)
  h = h + b_ref[...]
  h_ref[...] = jnp.maximum(h, 0.0)
  dv_ref[...] = jnp.broadcast_to(dinv, (_BN, D))


def _dense2_body(acc_ref, dv_ref, h_ref, wl_ref, wr_ref, b_ref, o_ref):
  agg = (acc_ref[0] + acc_ref[1]) * dv_ref[...]
  o = lax.dot_general(agg, wl_ref[...], (((1,), (1,)), ((), ())),
                      preferred_element_type=jnp.float32)
  o = o + lax.dot_general(h_ref[...], wr_ref[...], (((1,), (1,)), ((), ())),
                          preferred_element_type=jnp.float32)
  o_ref[...] = o + b_ref[...]


def _dense1(acc, dg, x, wl, wr, b):
  grid = (N // _BN,)
  return pl.pallas_call(
      _dense1_body,
      grid=grid,
      in_specs=[
          pl.BlockSpec((NC, _BN, D), lambda i: (0, i, 0)),
          pl.BlockSpec((NC, _BN, DG), lambda i: (0, i, 0)),
          pl.BlockSpec((_BN, D), lambda i: (i, 0)),
          pl.BlockSpec((D, D), lambda i: (0, 0)),
          pl.BlockSpec((D, D), lambda i: (0, 0)),
          pl.BlockSpec((1, D), lambda i: (0, 0)),
      ],
      out_specs=[
          pl.BlockSpec((_BN, D), lambda i: (i, 0)),
          pl.BlockSpec((_BN, D), lambda i: (i, 0)),
      ],
      out_shape=[
          jax.ShapeDtypeStruct((N, D), jnp.float32),
          jax.ShapeDtypeStruct((N, D), jnp.float32),
      ],
  )(acc, dg, x, wl, wr, b)


def _dense2(acc, dv, h, wl, wr, b):
  grid = (N // _BN,)
  return pl.pallas_call(
      _dense2_body,
      grid=grid,
      in_specs=[
          pl.BlockSpec((NC, _BN, D), lambda i: (0, i, 0)),
          pl.BlockSpec((_BN, D), lambda i: (i, 0)),
          pl.BlockSpec((_BN, D), lambda i: (i, 0)),
          pl.BlockSpec((D, D), lambda i: (0, 0)),
          pl.BlockSpec((D, D), lambda i: (0, 0)),
          pl.BlockSpec((1, D), lambda i: (0, 0)),
      ],
      out_specs=pl.BlockSpec((_BN, D), lambda i: (i, 0)),
      out_shape=jax.ShapeDtypeStruct((N, D), jnp.float32),
  )(acc, dv, h, wl, wr, b)


def kernel(x, edge_index, W1_l, W1_r, b1, W2_l, W2_r, b2):
  src = edge_index[0].reshape(NW, NB, SB, C)
  dst = edge_index[1].reshape(NW, NB, SB, C)
  zeros_d = jnp.zeros((RPT, D), jnp.float32)
  zeros_dg = jnp.zeros((RPT, DG), jnp.float32)
  ones_c = jnp.ones((C, DG), jnp.float32)

  acc1, dg = _segsum_l1(x, src, dst, zeros_d, ones_c, zeros_dg)
  h, dv = _dense1(acc1, dg, x, W1_l, W1_r, b1.reshape(1, D))
  acc2 = _segsum_l2(h, src, dst, zeros_d)
  out = _dense2(acc2, dv, h, W2_l, W2_r, b2.reshape(1, D))
  return out
